# initial kernel scaffold (unmeasured)
import jax
import jax.numpy as jnp
from jax import lax
from jax.experimental import pallas as pl
from jax.experimental.pallas import tpu as pltpu

N_DEV = 4


def kernel(x, w_mat):
    m_total, k_loc = x.shape
    _, n = w_mat.shape
    mc = m_total // N_DEV

    def body(
        x_ref,
        w_ref,
        out_ref,
        comm_ref,
        amax_send_ref,
        amax_recv_ref,
        send_sems,
        recv_sems,
        amax_send_sems,
        amax_recv_sems,
    ):
        r = lax.axis_index("i")
        left = jnp.mod(r - 1, N_DEV)
        right = jnp.mod(r + 1, N_DEV)

        barrier_sem = pltpu.get_barrier_semaphore()
        for nbr in (left, right):
            pl.semaphore_signal(
                barrier_sem,
                inc=1,
                device_id=(nbr,),
                device_id_type=pl.DeviceIdType.MESH,
            )
        pl.semaphore_wait(barrier_sem, 2)

        def partial_chunk(j):
            return jnp.dot(
                x_ref[pl.ds(j * mc, mc), :],
                w_ref[...],
                preferred_element_type=jnp.float32,
            )

        for s in range(N_DEV - 1):
            j = jnp.mod(r - s - 1, N_DEV)
            if s == 0:
                src_slot = N_DEV - 1
                comm_ref[src_slot] = partial_chunk(j).astype(jnp.bfloat16)
            else:
                src_slot = s - 1
                comm_ref[src_slot] = (
                    comm_ref[src_slot].astype(jnp.float32) + partial_chunk(j)
                ).astype(jnp.bfloat16)
            rdma = pltpu.make_async_remote_copy(
                src_ref=comm_ref.at[src_slot],
                dst_ref=comm_ref.at[s],
                send_sem=send_sems.at[s],
                recv_sem=recv_sems.at[s],
                device_id=(right,),
                device_id_type=pl.DeviceIdType.MESH,
            )
            rdma.start()
            rdma.wait()

        y = comm_ref[N_DEV - 2].astype(jnp.float32) + partial_chunk(r)
        y = jnp.maximum(y, 0.0)

        m_loc = jnp.max(y)
        amax_send_ref[...] = jnp.full((8, 128), m_loc, jnp.float32)
        rdmas = []
        for off in (1, 2, 3):
            tgt = jnp.mod(r + off, N_DEV)
            a_rdma = pltpu.make_async_remote_copy(
                src_ref=amax_send_ref,
                dst_ref=amax_recv_ref.at[off],
                send_sem=amax_send_sems.at[off - 1],
                recv_sem=amax_recv_sems.at[off - 1],
                device_id=(tgt,),
                device_id_type=pl.DeviceIdType.MESH,
            )
            a_rdma.start()
            rdmas.append(a_rdma)
        for a_rdma in rdmas:
            a_rdma.wait_send()
            a_rdma.wait_recv()

        amax_g = jnp.maximum(
            jnp.maximum(m_loc, amax_recv_ref[1, 0, 0]),
            jnp.maximum(amax_recv_ref[2, 0, 0], amax_recv_ref[3, 0, 0]),
        )

        scale = amax_g / 127.0
        q = jnp.clip(jnp.round(y * (127.0 / amax_g)), -127.0, 127.0)
        out_ref[...] = q * scale

    return pl.pallas_call(
        body,
        out_shape=jax.ShapeDtypeStruct((mc, n), jnp.float32),
        in_specs=[
            pl.BlockSpec(memory_space=pltpu.VMEM),
            pl.BlockSpec(memory_space=pltpu.VMEM),
        ],
        out_specs=pl.BlockSpec(memory_space=pltpu.VMEM),
        scratch_shapes=[
            pltpu.VMEM((N_DEV, mc, n), jnp.bfloat16),
            pltpu.VMEM((8, 128), jnp.float32),
            pltpu.VMEM((N_DEV, 8, 128), jnp.float32),
            pltpu.SemaphoreType.DMA((N_DEV - 1,)),
            pltpu.SemaphoreType.DMA((N_DEV - 1,)),
            pltpu.SemaphoreType.DMA((N_DEV - 1,)),
            pltpu.SemaphoreType.DMA((N_DEV - 1,)),
        ],
        compiler_params=pltpu.CompilerParams(collective_id=0),
    )(x, w_mat)


# baseline (device time: 181942 ns/iter reference)
import jax
import jax.numpy as jnp
from jax import lax
from jax.experimental import pallas as pl
from jax.experimental.pallas import tpu as pltpu

N_DEV = 4


def kernel(x, w_mat):
    x = x.astype(jnp.bfloat16)
    w_mat = w_mat.astype(jnp.bfloat16)
    m_total, k_loc = x.shape
    _, n = w_mat.shape
    mc = m_total // N_DEV

    def body(
        x_ref,
        w_ref,
        out_ref,
        comm_ref,
        amax_send_ref,
        amax_recv_ref,
        send_sems,
        recv_sems,
        amax_send_sems,
        amax_recv_sems,
    ):
        r = lax.axis_index("i")
        left = jnp.mod(r - 1, N_DEV)
        right = jnp.mod(r + 1, N_DEV)

        barrier_sem = pltpu.get_barrier_semaphore()
        for nbr in (left, right):
            pl.semaphore_signal(
                barrier_sem,
                inc=1,
                device_id=(nbr,),
                device_id_type=pl.DeviceIdType.MESH,
            )
        pl.semaphore_wait(barrier_sem, 2)

        def partial_chunk(j):
            return jnp.dot(
                x_ref[pl.ds(j * mc, mc), :],
                w_ref[...],
                preferred_element_type=jnp.float32,
            )

        for s in range(N_DEV - 1):
            j = jnp.mod(r - s - 1, N_DEV)
            if s == 0:
                src_slot = N_DEV - 1
                comm_ref[src_slot] = partial_chunk(j).astype(jnp.bfloat16)
            else:
                src_slot = s - 1
                comm_ref[src_slot] = (
                    comm_ref[src_slot].astype(jnp.float32) + partial_chunk(j)
                ).astype(jnp.bfloat16)
            rdma = pltpu.make_async_remote_copy(
                src_ref=comm_ref.at[src_slot],
                dst_ref=comm_ref.at[s],
                send_sem=send_sems.at[s],
                recv_sem=recv_sems.at[s],
                device_id=(right,),
                device_id_type=pl.DeviceIdType.MESH,
            )
            rdma.start()
            rdma.wait()

        y = comm_ref[N_DEV - 2].astype(jnp.float32) + partial_chunk(r)
        y = jnp.maximum(y, 0.0)

        m_loc = jnp.max(y)
        amax_send_ref[...] = jnp.full((8, 128), m_loc, jnp.float32)
        rdmas = []
        for off in (1, 2, 3):
            tgt = jnp.mod(r + off, N_DEV)
            a_rdma = pltpu.make_async_remote_copy(
                src_ref=amax_send_ref,
                dst_ref=amax_recv_ref.at[off],
                send_sem=amax_send_sems.at[off - 1],
                recv_sem=amax_recv_sems.at[off - 1],
                device_id=(tgt,),
                device_id_type=pl.DeviceIdType.MESH,
            )
            a_rdma.start()
            rdmas.append(a_rdma)
        for a_rdma in rdmas:
            a_rdma.wait_send()
            a_rdma.wait_recv()

        amax_g = jnp.maximum(
            jnp.maximum(m_loc, amax_recv_ref[1, 0, 0]),
            jnp.maximum(amax_recv_ref[2, 0, 0], amax_recv_ref[3, 0, 0]),
        )

        scale = amax_g / 127.0
        q = jnp.clip(jnp.round(y * (127.0 / amax_g)), -127.0, 127.0)
        out_ref[...] = q * scale

    return pl.pallas_call(
        body,
        out_shape=jax.ShapeDtypeStruct((mc, n), jnp.float32),
        in_specs=[
            pl.BlockSpec(memory_space=pltpu.VMEM),
            pl.BlockSpec(memory_space=pltpu.VMEM),
        ],
        out_specs=pl.BlockSpec(memory_space=pltpu.VMEM),
        scratch_shapes=[
            pltpu.VMEM((N_DEV, mc, n), jnp.bfloat16),
            pltpu.VMEM((8, 128), jnp.float32),
            pltpu.VMEM((N_DEV, 8, 128), jnp.float32),
            pltpu.SemaphoreType.DMA((N_DEV - 1,)),
            pltpu.SemaphoreType.DMA((N_DEV - 1,)),
            pltpu.SemaphoreType.DMA((N_DEV - 1,)),
            pltpu.SemaphoreType.DMA((N_DEV - 1,)),
        ],
        compiler_params=pltpu.CompilerParams(collective_id=0),
    )(x, w_mat)


# device time: 103560 ns/iter; 1.7569x vs baseline; 1.7569x over previous
import jax
import jax.numpy as jnp
from jax import lax
from jax.experimental import pallas as pl
from jax.experimental.pallas import tpu as pltpu

N_DEV = 4


def kernel(x, w_mat):
    x = x.astype(jnp.bfloat16)
    w_mat = w_mat.astype(jnp.bfloat16)
    m_total, k_loc = x.shape
    _, n = w_mat.shape
    mc = m_total // N_DEV
    nh = n // 2

    def body(
        x_ref,
        w_ref,
        out_ref,
        comm_r_ref,
        comm_l_ref,
        amax_send_ref,
        amax_recv_ref,
        send_sems_r,
        recv_sems_r,
        send_sems_l,
        recv_sems_l,
        amax_send_sems,
        amax_recv_sems,
    ):
        r = lax.axis_index("i")
        left = jnp.mod(r - 1, N_DEV)
        right = jnp.mod(r + 1, N_DEV)

        barrier_sem = pltpu.get_barrier_semaphore()
        for nbr in (left, right):
            pl.semaphore_signal(
                barrier_sem,
                inc=1,
                device_id=(nbr,),
                device_id_type=pl.DeviceIdType.MESH,
            )
        pl.semaphore_wait(barrier_sem, 2)

        def partial(j, lo, width):
            return jnp.dot(
                x_ref[pl.ds(j * mc, mc), :],
                w_ref[:, lo : lo + width],
                preferred_element_type=jnp.float32,
            )

        def hop(s, src_slot_fn):
            rr = pltpu.make_async_remote_copy(
                src_ref=comm_r_ref.at[src_slot_fn],
                dst_ref=comm_r_ref.at[s],
                send_sem=send_sems_r.at[s],
                recv_sem=recv_sems_r.at[s],
                device_id=(right,),
                device_id_type=pl.DeviceIdType.MESH,
            )
            rl = pltpu.make_async_remote_copy(
                src_ref=comm_l_ref.at[src_slot_fn],
                dst_ref=comm_l_ref.at[s],
                send_sem=send_sems_l.at[s],
                recv_sem=recv_sems_l.at[s],
                device_id=(left,),
                device_id_type=pl.DeviceIdType.MESH,
            )
            rr.start()
            rl.start()
            return rr, rl

        comm_r_ref[3] = partial(jnp.mod(r - 1, N_DEV), 0, nh).astype(jnp.bfloat16)
        comm_l_ref[3] = partial(jnp.mod(r + 1, N_DEV), nh, nh).astype(jnp.bfloat16)
        rr0, rl0 = hop(0, 3)
        p1 = partial(jnp.mod(r + 2, N_DEV), 0, n)
        rr0.wait()
        rl0.wait()

        comm_r_ref[0] = (
            comm_r_ref[0].astype(jnp.float32) + p1[:, :nh]
        ).astype(jnp.bfloat16)
        comm_l_ref[0] = (
            comm_l_ref[0].astype(jnp.float32) + p1[:, nh:]
        ).astype(jnp.bfloat16)
        rr1, rl1 = hop(1, 0)
        p2a = partial(jnp.mod(r + 1, N_DEV), 0, nh)
        p2b = partial(jnp.mod(r - 1, N_DEV), nh, nh)
        rr1.wait()
        rl1.wait()

        comm_r_ref[1] = (comm_r_ref[1].astype(jnp.float32) + p2a).astype(
            jnp.bfloat16
        )
        comm_l_ref[1] = (comm_l_ref[1].astype(jnp.float32) + p2b).astype(
            jnp.bfloat16
        )
        rr2, rl2 = hop(2, 1)
        pown = partial(r, 0, n)
        rr2.wait()
        rl2.wait()

        ya = jnp.maximum(comm_r_ref[2].astype(jnp.float32) + pown[:, :nh], 0.0)
        yb = jnp.maximum(comm_l_ref[2].astype(jnp.float32) + pown[:, nh:], 0.0)

        m_loc = jnp.maximum(jnp.max(ya), jnp.max(yb))
        amax_send_ref[...] = jnp.full((8, 128), m_loc, jnp.float32)
        rdmas = []
        for off in (1, 2, 3):
            tgt = jnp.mod(r + off, N_DEV)
            a_rdma = pltpu.make_async_remote_copy(
                src_ref=amax_send_ref,
                dst_ref=amax_recv_ref.at[off],
                send_sem=amax_send_sems.at[off - 1],
                recv_sem=amax_recv_sems.at[off - 1],
                device_id=(tgt,),
                device_id_type=pl.DeviceIdType.MESH,
            )
            a_rdma.start()
            rdmas.append(a_rdma)
        for a_rdma in rdmas:
            a_rdma.wait_send()
            a_rdma.wait_recv()

        amax_g = jnp.maximum(
            jnp.maximum(m_loc, amax_recv_ref[1, 0, 0]),
            jnp.maximum(amax_recv_ref[2, 0, 0], amax_recv_ref[3, 0, 0]),
        )

        scale = amax_g / 127.0
        inv = 127.0 / amax_g
        out_ref[:, :nh] = jnp.clip(jnp.round(ya * inv), -127.0, 127.0) * scale
        out_ref[:, nh:] = jnp.clip(jnp.round(yb * inv), -127.0, 127.0) * scale

    return pl.pallas_call(
        body,
        out_shape=jax.ShapeDtypeStruct((mc, n), jnp.float32),
        in_specs=[
            pl.BlockSpec(memory_space=pltpu.VMEM),
            pl.BlockSpec(memory_space=pltpu.VMEM),
        ],
        out_specs=pl.BlockSpec(memory_space=pltpu.VMEM),
        scratch_shapes=[
            pltpu.VMEM((N_DEV, mc, nh), jnp.bfloat16),
            pltpu.VMEM((N_DEV, mc, nh), jnp.bfloat16),
            pltpu.VMEM((8, 128), jnp.float32),
            pltpu.VMEM((N_DEV, 8, 128), jnp.float32),
            pltpu.SemaphoreType.DMA((N_DEV - 1,)),
            pltpu.SemaphoreType.DMA((N_DEV - 1,)),
            pltpu.SemaphoreType.DMA((N_DEV - 1,)),
            pltpu.SemaphoreType.DMA((N_DEV - 1,)),
            pltpu.SemaphoreType.DMA((N_DEV - 1,)),
            pltpu.SemaphoreType.DMA((N_DEV - 1,)),
        ],
        compiler_params=pltpu.CompilerParams(collective_id=0),
    )(x, w_mat)


# device time: 95245 ns/iter; 1.9103x vs baseline; 1.0873x over previous
import jax
import jax.numpy as jnp
from jax import lax
from jax.experimental import pallas as pl
from jax.experimental.pallas import tpu as pltpu

N_DEV = 4
N_TILE = 2


def kernel(x, w_mat):
    x = x.astype(jnp.bfloat16)
    w_mat = w_mat.astype(jnp.bfloat16)
    m_total, k_loc = x.shape
    _, n = w_mat.shape
    mc = m_total // N_DEV
    nh = n // 2
    tw = nh // N_TILE

    def body(
        x_ref,
        w_ref,
        out_ref,
        comm_r_ref,
        comm_l_ref,
        amax_send_ref,
        amax_recv_ref,
        send_sems_r,
        recv_sems_r,
        send_sems_l,
        recv_sems_l,
        amax_send_sems,
        amax_recv_sems,
    ):
        r = lax.axis_index("i")
        left = jnp.mod(r - 1, N_DEV)
        right = jnp.mod(r + 1, N_DEV)

        barrier_sem = pltpu.get_barrier_semaphore()
        for nbr in (left, right):
            pl.semaphore_signal(
                barrier_sem,
                inc=1,
                device_id=(nbr,),
                device_id_type=pl.DeviceIdType.MESH,
            )
        pl.semaphore_wait(barrier_sem, 2)

        def partial(j, lo, width):
            return jnp.dot(
                x_ref[pl.ds(j * mc, mc), :],
                w_ref[:, lo : lo + width],
                preferred_element_type=jnp.float32,
            )

        def send_tile(comm, ssems, rsems, src_slot, s, t, dev):
            rd = pltpu.make_async_remote_copy(
                src_ref=comm.at[src_slot, :, pl.ds(t * tw, tw)],
                dst_ref=comm.at[s, :, pl.ds(t * tw, tw)],
                send_sem=ssems.at[s, t],
                recv_sem=rsems.at[s, t],
                device_id=(dev,),
                device_id_type=pl.DeviceIdType.MESH,
            )
            rd.start()
            return rd

        rr = [[None] * N_TILE for _ in range(3)]
        rl = [[None] * N_TILE for _ in range(3)]

        jr0 = jnp.mod(r - 1, N_DEV)
        jl0 = jnp.mod(r + 1, N_DEV)
        for t in range(N_TILE):
            ts = pl.ds(t * tw, tw)
            comm_r_ref[3, :, ts] = partial(jr0, t * tw, tw).astype(jnp.bfloat16)
            rr[0][t] = send_tile(
                comm_r_ref, send_sems_r, recv_sems_r, 3, 0, t, right
            )
            comm_l_ref[3, :, ts] = partial(jl0, nh + t * tw, tw).astype(
                jnp.bfloat16
            )
            rl[0][t] = send_tile(
                comm_l_ref, send_sems_l, recv_sems_l, 3, 0, t, left
            )

        p1 = partial(jnp.mod(r + 2, N_DEV), 0, n)

        for s, (pa, pb) in (
            (1, (lambda: p1[:, :nh], lambda: p1[:, nh:])),
            (2, (lambda: p2a, lambda: p2b)),
        ):
            for t in range(N_TILE):
                ts = pl.ds(t * tw, tw)
                rr[s - 1][t].wait_recv()
                comm_r_ref[s - 1, :, ts] = (
                    comm_r_ref[s - 1, :, ts].astype(jnp.float32)
                    + pa()[:, t * tw : (t + 1) * tw]
                ).astype(jnp.bfloat16)
                rr[s][t] = send_tile(
                    comm_r_ref, send_sems_r, recv_sems_r, s - 1, s, t, right
                )
                rl[s - 1][t].wait_recv()
                comm_l_ref[s - 1, :, ts] = (
                    comm_l_ref[s - 1, :, ts].astype(jnp.float32)
                    + pb()[:, t * tw : (t + 1) * tw]
                ).astype(jnp.bfloat16)
                rl[s][t] = send_tile(
                    comm_l_ref, send_sems_l, recv_sems_l, s - 1, s, t, left
                )
            if s == 1:
                p2a = partial(jnp.mod(r + 1, N_DEV), 0, nh)
                p2b = partial(jnp.mod(r - 1, N_DEV), nh, nh)

        pown = partial(r, 0, n)

        for t in range(N_TILE):
            rr[2][t].wait_recv()
            rl[2][t].wait_recv()
        for s in range(3):
            for t in range(N_TILE):
                rr[s][t].wait_send()
                rl[s][t].wait_send()

        ya = jnp.maximum(comm_r_ref[2].astype(jnp.float32) + pown[:, :nh], 0.0)
        yb = jnp.maximum(comm_l_ref[2].astype(jnp.float32) + pown[:, nh:], 0.0)

        m_loc = jnp.maximum(jnp.max(ya), jnp.max(yb))
        amax_send_ref[...] = jnp.full((8, 128), m_loc, jnp.float32)
        rdmas = []
        for off in (1, 2, 3):
            tgt = jnp.mod(r + off, N_DEV)
            a_rdma = pltpu.make_async_remote_copy(
                src_ref=amax_send_ref,
                dst_ref=amax_recv_ref.at[off],
                send_sem=amax_send_sems.at[off - 1],
                recv_sem=amax_recv_sems.at[off - 1],
                device_id=(tgt,),
                device_id_type=pl.DeviceIdType.MESH,
            )
            a_rdma.start()
            rdmas.append(a_rdma)
        for a_rdma in rdmas:
            a_rdma.wait_send()
            a_rdma.wait_recv()

        amax_g = jnp.maximum(
            jnp.maximum(m_loc, amax_recv_ref[1, 0, 0]),
            jnp.maximum(amax_recv_ref[2, 0, 0], amax_recv_ref[3, 0, 0]),
        )

        scale = amax_g / 127.0
        inv = 127.0 / amax_g
        out_ref[:, :nh] = jnp.clip(jnp.round(ya * inv), -127.0, 127.0) * scale
        out_ref[:, nh:] = jnp.clip(jnp.round(yb * inv), -127.0, 127.0) * scale

    return pl.pallas_call(
        body,
        out_shape=jax.ShapeDtypeStruct((mc, n), jnp.float32),
        in_specs=[
            pl.BlockSpec(memory_space=pltpu.VMEM),
            pl.BlockSpec(memory_space=pltpu.VMEM),
        ],
        out_specs=pl.BlockSpec(memory_space=pltpu.VMEM),
        scratch_shapes=[
            pltpu.VMEM((N_DEV, mc, nh), jnp.bfloat16),
            pltpu.VMEM((N_DEV, mc, nh), jnp.bfloat16),
            pltpu.VMEM((8, 128), jnp.float32),
            pltpu.VMEM((N_DEV, 8, 128), jnp.float32),
            pltpu.SemaphoreType.DMA((3, N_TILE)),
            pltpu.SemaphoreType.DMA((3, N_TILE)),
            pltpu.SemaphoreType.DMA((3, N_TILE)),
            pltpu.SemaphoreType.DMA((3, N_TILE)),
            pltpu.SemaphoreType.DMA((N_DEV - 1,)),
            pltpu.SemaphoreType.DMA((N_DEV - 1,)),
        ],
        compiler_params=pltpu.CompilerParams(collective_id=0),
    )(x, w_mat)


# device time: 93482 ns/iter; 1.9463x vs baseline; 1.0189x over previous
import jax
import jax.numpy as jnp
from jax import lax
from jax.experimental import pallas as pl
from jax.experimental.pallas import tpu as pltpu

N_DEV = 4
N_TILE = 2


def kernel(x, w_mat):
    m_total, k_loc = x.shape
    _, n = w_mat.shape
    mc = m_total // N_DEV
    nh = n // 2
    tw = nh // N_TILE

    def body(
        x_ref,
        w_ref,
        out_ref,
        comm_r_ref,
        comm_l_ref,
        w_bf16_ref,
        stage_ref,
        xb_ref,
        amax_send_ref,
        amax_recv_ref,
        send_sems_r,
        recv_sems_r,
        send_sems_l,
        recv_sems_l,
        stage_sems,
        amax_send_sems,
        amax_recv_sems,
    ):
        r = lax.axis_index("i")
        left = jnp.mod(r - 1, N_DEV)
        right = jnp.mod(r + 1, N_DEV)

        barrier_sem = pltpu.get_barrier_semaphore()
        for nbr in (left, right):
            pl.semaphore_signal(
                barrier_sem,
                inc=1,
                device_id=(nbr,),
                device_id_type=pl.DeviceIdType.MESH,
            )
        pl.semaphore_wait(barrier_sem, 2)

        hk = mc // 2
        cps = {}

        def issue(i, src):
            cp = pltpu.make_async_copy(
                src, stage_ref.at[i % 2], stage_sems.at[i % 2]
            )
            cp.start()
            cps[i] = cp

        def issue_x(i, j, rh):
            issue(i, x_ref.at[pl.ds(j * mc + rh * hk, hk), :])

        def conv(i, dst):
            cps[i].wait()
            dst[...] = stage_ref[i % 2].astype(jnp.bfloat16)

        def dot_b(b, lo, width):
            return jnp.dot(
                xb_ref[b],
                w_bf16_ref[:, lo : lo + width],
                preferred_element_type=jnp.float32,
            )

        def send_tile(comm, ssems, rsems, src_slot, s, t, dev):
            rd = pltpu.make_async_remote_copy(
                src_ref=comm.at[src_slot, :, pl.ds(t * tw, tw)],
                dst_ref=comm.at[s, :, pl.ds(t * tw, tw)],
                send_sem=ssems.at[s, t],
                recv_sem=rsems.at[s, t],
                device_id=(dev,),
                device_id_type=pl.DeviceIdType.MESH,
            )
            rd.start()
            return rd

        jm1 = jnp.mod(r - 1, N_DEV)
        jp1 = jnp.mod(r + 1, N_DEV)
        jp2 = jnp.mod(r + 2, N_DEV)
        issue(0, w_ref.at[0:hk, 0:nh])
        issue(1, w_ref.at[hk : 2 * hk, 0:nh])
        conv(0, w_bf16_ref.at[0:hk, 0:nh])
        issue(2, w_ref.at[0:hk, nh:n])
        conv(1, w_bf16_ref.at[hk : 2 * hk, 0:nh])
        issue(3, w_ref.at[hk : 2 * hk, nh:n])
        conv(2, w_bf16_ref.at[0:hk, nh:n])
        issue_x(4, jm1, 0)
        conv(3, w_bf16_ref.at[hk : 2 * hk, nh:n])
        issue_x(5, jm1, 1)
        conv(4, xb_ref.at[0, 0:hk, :])
        issue_x(6, jp1, 0)
        conv(5, xb_ref.at[0, hk:mc, :])
        issue_x(7, jp1, 1)

        rr = [[None] * N_TILE for _ in range(3)]
        rl = [[None] * N_TILE for _ in range(3)]

        for t in range(N_TILE):
            ts = pl.ds(t * tw, tw)
            comm_r_ref[2, :, ts] = dot_b(0, t * tw, tw).astype(jnp.bfloat16)
            rr[0][t] = send_tile(
                comm_r_ref, send_sems_r, recv_sems_r, 2, 0, t, right
            )
        conv(6, xb_ref.at[1, 0:hk, :])
        issue_x(8, jp2, 0)
        conv(7, xb_ref.at[1, hk:mc, :])
        issue_x(9, jp2, 1)
        for t in range(N_TILE):
            ts = pl.ds(t * tw, tw)
            comm_l_ref[2, :, ts] = dot_b(1, nh + t * tw, tw).astype(jnp.bfloat16)
            rl[0][t] = send_tile(
                comm_l_ref, send_sems_l, recv_sems_l, 2, 0, t, left
            )

        conv(8, xb_ref.at[0, 0:hk, :])
        issue_x(10, jm1, 0)
        conv(9, xb_ref.at[0, hk:mc, :])
        issue_x(11, jm1, 1)
        p1 = dot_b(0, 0, n)

        for t in range(N_TILE):
            ts = pl.ds(t * tw, tw)
            cs = slice(t * tw, (t + 1) * tw)
            rr[0][t].wait_recv()
            comm_r_ref[0, :, ts] = (
                comm_r_ref[0, :, ts].astype(jnp.float32) + p1[:, :nh][:, cs]
            ).astype(jnp.bfloat16)
            rr[1][t] = send_tile(
                comm_r_ref, send_sems_r, recv_sems_r, 0, 1, t, right
            )
            rl[0][t].wait_recv()
            comm_l_ref[0, :, ts] = (
                comm_l_ref[0, :, ts].astype(jnp.float32) + p1[:, nh:][:, cs]
            ).astype(jnp.bfloat16)
            rl[1][t] = send_tile(
                comm_l_ref, send_sems_l, recv_sems_l, 0, 1, t, left
            )

        p2a = dot_b(1, 0, nh)
        conv(10, xb_ref.at[0, 0:hk, :])
        issue_x(12, r, 0)
        conv(11, xb_ref.at[0, hk:mc, :])
        issue_x(13, r, 1)
        p2b = dot_b(0, nh, nh)

        for t in range(N_TILE):
            ts = pl.ds(t * tw, tw)
            cs = slice(t * tw, (t + 1) * tw)
            rr[1][t].wait_recv()
            comm_r_ref[1, :, ts] = (
                comm_r_ref[1, :, ts].astype(jnp.float32) + p2a[:, cs]
            ).astype(jnp.bfloat16)
            rr[2][t] = send_tile(
                comm_r_ref, send_sems_r, recv_sems_r, 1, 2, t, right
            )
            rl[1][t].wait_recv()
            comm_l_ref[1, :, ts] = (
                comm_l_ref[1, :, ts].astype(jnp.float32) + p2b[:, cs]
            ).astype(jnp.bfloat16)
            rl[2][t] = send_tile(
                comm_l_ref, send_sems_l, recv_sems_l, 1, 2, t, left
            )

        conv(12, xb_ref.at[1, 0:hk, :])
        conv(13, xb_ref.at[1, hk:mc, :])
        pown = dot_b(1, 0, n)

        for t in range(N_TILE):
            rr[2][t].wait_recv()
            rl[2][t].wait_recv()
        for s in range(3):
            for t in range(N_TILE):
                rr[s][t].wait_send()
                rl[s][t].wait_send()

        ya = jnp.maximum(comm_r_ref[2].astype(jnp.float32) + pown[:, :nh], 0.0)
        yb = jnp.maximum(comm_l_ref[2].astype(jnp.float32) + pown[:, nh:], 0.0)

        m_loc = jnp.maximum(jnp.max(ya), jnp.max(yb))
        amax_send_ref[...] = jnp.full((8, 128), m_loc, jnp.float32)
        rdmas = []
        for off in (1, 2, 3):
            tgt = jnp.mod(r + off, N_DEV)
            a_rdma = pltpu.make_async_remote_copy(
                src_ref=amax_send_ref,
                dst_ref=amax_recv_ref.at[off],
                send_sem=amax_send_sems.at[off - 1],
                recv_sem=amax_recv_sems.at[off - 1],
                device_id=(tgt,),
                device_id_type=pl.DeviceIdType.MESH,
            )
            a_rdma.start()
            rdmas.append(a_rdma)
        for a_rdma in rdmas:
            a_rdma.wait_send()
            a_rdma.wait_recv()

        amax_g = jnp.maximum(
            jnp.maximum(m_loc, amax_recv_ref[1, 0, 0]),
            jnp.maximum(amax_recv_ref[2, 0, 0], amax_recv_ref[3, 0, 0]),
        )

        scale = amax_g / 127.0
        inv = 127.0 / amax_g
        out_ref[:, :nh] = (
            jnp.clip(jnp.round(ya * inv), -127.0, 127.0) * scale
        ).astype(jnp.bfloat16)
        out_ref[:, nh:] = (
            jnp.clip(jnp.round(yb * inv), -127.0, 127.0) * scale
        ).astype(jnp.bfloat16)

    return pl.pallas_call(
        body,
        out_shape=jax.ShapeDtypeStruct((mc, n), jnp.bfloat16),
        in_specs=[
            pl.BlockSpec(memory_space=pl.ANY),
            pl.BlockSpec(memory_space=pl.ANY),
        ],
        out_specs=pl.BlockSpec(memory_space=pltpu.VMEM),
        scratch_shapes=[
            pltpu.VMEM((3, mc, nh), jnp.bfloat16),
            pltpu.VMEM((3, mc, nh), jnp.bfloat16),
            pltpu.VMEM((k_loc, n), jnp.bfloat16),
            pltpu.VMEM((2, mc // 2, k_loc), jnp.float32),
            pltpu.VMEM((2, mc, k_loc), jnp.bfloat16),
            pltpu.VMEM((8, 128), jnp.float32),
            pltpu.VMEM((N_DEV, 8, 128), jnp.float32),
            pltpu.SemaphoreType.DMA((3, N_TILE)),
            pltpu.SemaphoreType.DMA((3, N_TILE)),
            pltpu.SemaphoreType.DMA((3, N_TILE)),
            pltpu.SemaphoreType.DMA((3, N_TILE)),
            pltpu.SemaphoreType.DMA((2,)),
            pltpu.SemaphoreType.DMA((N_DEV - 1,)),
            pltpu.SemaphoreType.DMA((N_DEV - 1,)),
        ],
        compiler_params=pltpu.CompilerParams(collective_id=0),
    )(x, w_mat)
